# baseline (speedup 1.0000x reference)
import functools

import jax
import jax.numpy as jnp
from jax import lax
from jax.experimental import pallas as pl
from jax.experimental.pallas import tpu as pltpu
from jax.experimental.pallas import tpu_sc as plsc


def kernel(x, table):
    B, S = x.shape
    V, D = table.shape
    N = B * S

    NW = 32
    G = 128
    K = 8
    T = N // (NW * G * K)
    n_groups = T * K

    idx = x.reshape(NW, T, K * G).astype(jnp.int32)

    mesh = plsc.VectorSubcoreMesh(core_axis_name="c", subcore_axis_name="s")

    @functools.partial(
        pl.kernel,
        mesh=mesh,
        out_type=jax.ShapeDtypeStruct((NW * T, K * G, D), jnp.float32),
        compiler_params=pltpu.CompilerParams(use_tc_tiling_on_sc=False),
        scratch_types=[
            pltpu.VMEM((T, K * G), jnp.int32),
            pltpu.VMEM((2, K * G, D), jnp.float32),
            pltpu.SemaphoreType.DMA,
            pltpu.SemaphoreType.DMA,
        ],
    )
    def emb(idx_hbm, table_hbm, out_hbm, idx_v, rows_v, gsem, osem):
        wid = lax.axis_index("s") * 2 + lax.axis_index("c")
        pltpu.sync_copy(idx_hbm.at[wid], idx_v)
        wbase = wid * T

        def body(t, carry):
            p = lax.rem(t, 2)

            @pl.when(t >= 2)
            def _wait_prev_out():
                pltpu.make_async_copy(rows_v.at[p], out_hbm.at[wbase], osem).wait()

            pltpu.async_copy(
                table_hbm.at[idx_v.at[t]], rows_v.at[p], gsem
            ).wait()
            pltpu.async_copy(rows_v.at[p], out_hbm.at[wbase + t], osem)
            return carry

        lax.fori_loop(0, T, body, 0)
        pltpu.make_async_copy(rows_v.at[0], out_hbm.at[wbase], osem).wait()
        pltpu.make_async_copy(rows_v.at[1], out_hbm.at[wbase], osem).wait()

    out = emb(idx, table)
    return out.reshape(B, S, D)


# speedup vs baseline: 1.4018x; 1.4018x over previous
import functools

import jax
import jax.numpy as jnp
from jax import lax
from jax.experimental import pallas as pl
from jax.experimental.pallas import tpu as pltpu
from jax.experimental.pallas import tpu_sc as plsc


def kernel(x, table):
    B, S = x.shape
    V, D = table.shape

    NW = 32
    b_per_w = B // NW
    NB = 16
    T = b_per_w // NB

    idx = x.astype(jnp.int32)

    mesh = plsc.VectorSubcoreMesh(core_axis_name="c", subcore_axis_name="s")

    @functools.partial(
        pl.kernel,
        mesh=mesh,
        out_type=jax.ShapeDtypeStruct((B, S, D), jnp.float32),
        compiler_params=pltpu.CompilerParams(use_tc_tiling_on_sc=False),
        scratch_types=[
            pltpu.VMEM((b_per_w, S), jnp.int32),
            pltpu.VMEM((2, NB, S, D), jnp.float32),
            pltpu.SemaphoreType.DMA,
            pltpu.SemaphoreType.DMA,
        ],
    )
    def emb(idx_hbm, table_hbm, out_hbm, idx_v, rows_v, gsem, osem):
        wid = lax.axis_index("s") * 2 + lax.axis_index("c")
        wb = wid * b_per_w
        pltpu.sync_copy(idx_hbm.at[pl.ds(wb, b_per_w)], idx_v)

        def body(t, carry):
            p = lax.rem(t, 2)

            @pl.when(t >= 2)
            def _wait_prev_out():
                pltpu.make_async_copy(
                    rows_v.at[p], out_hbm.at[pl.ds(wb, NB)], osem).wait()

            copies = [
                pltpu.async_copy(
                    table_hbm.at[idx_v.at[t * NB + j]], rows_v.at[p, j], gsem)
                for j in range(NB)
            ]
            for c in copies:
                c.wait()
            pltpu.async_copy(
                rows_v.at[p], out_hbm.at[pl.ds(wb + t * NB, NB)], osem)
            return carry

        lax.fori_loop(0, T, body, 0)
        pltpu.make_async_copy(rows_v.at[0], out_hbm.at[pl.ds(wb, NB)], osem).wait()
        pltpu.make_async_copy(rows_v.at[1], out_hbm.at[pl.ds(wb, NB)], osem).wait()

    return emb(idx, table)


# speedup vs baseline: 1.4872x; 1.0609x over previous
import functools

import jax
import jax.numpy as jnp
from jax import lax
from jax.experimental import pallas as pl
from jax.experimental.pallas import tpu as pltpu
from jax.experimental.pallas import tpu_sc as plsc


def kernel(x, table):
    B, S = x.shape
    V, D = table.shape

    NW = 32
    b_per_w = B // NW

    xt = jnp.swapaxes(x, 0, 1).astype(jnp.int32)

    mesh = plsc.VectorSubcoreMesh(core_axis_name="c", subcore_axis_name="s")

    @functools.partial(
        pl.kernel,
        mesh=mesh,
        out_type=jax.ShapeDtypeStruct((S, B, D), jnp.float32),
        compiler_params=pltpu.CompilerParams(use_tc_tiling_on_sc=False),
        scratch_types=[
            pltpu.VMEM((S, b_per_w), jnp.int32),
            pltpu.VMEM((2, b_per_w, D), jnp.float32),
            pltpu.SemaphoreType.DMA,
            pltpu.SemaphoreType.DMA,
        ],
    )
    def emb(idx_hbm, table_hbm, out_hbm, idx_v, rows_v, gsem, osem):
        wid = lax.axis_index("s") * 2 + lax.axis_index("c")
        wb = wid * b_per_w
        pltpu.sync_copy(idx_hbm.at[:, pl.ds(wb, b_per_w)], idx_v)

        def body(s, carry):
            p = lax.rem(s, 2)

            @pl.when(s >= 2)
            def _wait_prev_out():
                pltpu.make_async_copy(
                    rows_v.at[p], out_hbm.at[0, pl.ds(wb, b_per_w)], osem
                ).wait()

            pltpu.async_copy(
                table_hbm.at[idx_v.at[s]], rows_v.at[p], gsem).wait()
            pltpu.async_copy(
                rows_v.at[p], out_hbm.at[s, pl.ds(wb, b_per_w)], osem)
            return carry

        lax.fori_loop(0, S, body, 0)
        pltpu.make_async_copy(
            rows_v.at[0], out_hbm.at[0, pl.ds(wb, b_per_w)], osem).wait()
        pltpu.make_async_copy(
            rows_v.at[1], out_hbm.at[0, pl.ds(wb, b_per_w)], osem).wait()

    out = emb(xt, table)
    return jnp.swapaxes(out, 0, 1)
